# Initial kernel scaffold; baseline (speedup 1.0000x reference)
import jax
import jax.numpy as jnp
from jax.experimental import pallas as pl


def kernel(user_emb, item_emb, hg_rows, hg_cols, hg_vals):
    raise NotImplementedError("write your pallas kernel here")


# speedup vs baseline: 5.8015x; 5.8015x over previous
import functools

import jax
import jax.numpy as jnp
from jax import lax
from jax.experimental import pallas as pl
from jax.experimental.pallas import tpu as pltpu
from jax.experimental.pallas import tpu_sc as plsc

N_USER = 20000
N_ITEM = 30000
N = N_USER + N_ITEM
NP = 51200
D = 64
HALF = 32
E = 800000
NC = 2
NS = 16
L = 16

SB = 80
SBC = 5
CHUNK = SB * SBC
SB_ROWS = E // SB
ROWS_PER_TILE = SB_ROWS // NS
CHUNKS_PER_TILE = ROWS_PER_TILE // SBC
ACC_ROWS_PER_TILE = NP // NS
ZROWS = 160


def _sc_body(ego, cols, rows, vals, out1, out2,
             col1_v, row1_v, idx_v, row_v, val_v, gath_v, zbuf_v, acc, sem):
    c = lax.axis_index("c")
    s = lax.axis_index("s")
    col_off = c * NP

    z16 = jnp.zeros((L,), jnp.float32)

    def _zb(j, carry):
        zbuf_v[j, pl.ds(0, L)] = z16
        zbuf_v[j, pl.ds(L, L)] = z16
        return carry

    lax.fori_loop(0, ZROWS, _zb, None)

    abase = s * ACC_ROWS_PER_TILE
    for z in range(ACC_ROWS_PER_TILE // ZROWS):
        pltpu.sync_copy(zbuf_v, acc.at[pl.ds(abase + z * ZROWS, ZROWS)])
    plsc.subcore_barrier()

    def layer(table, out, rezero):
        def chunk_body(g, carry):
            e0 = (s * ROWS_PER_TILE + g * SBC) * SB
            pltpu.sync_copy(cols.at[pl.ds(e0, CHUNK)], col1_v)
            pltpu.sync_copy(rows.at[pl.ds(e0, CHUNK)], row1_v)
            pltpu.sync_copy(vals.at[pl.ds(e0, CHUNK)], val_v)
            for j in range(SBC):
                for k in range(SB // L):
                    sl = pl.ds(k * L, L)
                    fl = pl.ds(j * SB + k * L, L)
                    idx_v[j, sl] = col1_v[fl] + col_off
                    row_v[j, sl] = row1_v[fl]
            descs = [
                pltpu.async_copy(table.at[idx_v.at[j]],
                                 gath_v.at[pl.ds(j * SB, SB)], sem)
                for j in range(SBC)
            ]
            for d in descs:
                d.wait()
            def _scale(t, cy):
                valvec = val_v[pl.ds(t * L, L)]
                for i in range(L):
                    r = t * L + i
                    v = valvec[i]
                    gath_v[r, pl.ds(0, L)] = gath_v[r, pl.ds(0, L)] * v
                    gath_v[r, pl.ds(L, L)] = gath_v[r, pl.ds(L, L)] * v
                return cy
            lax.fori_loop(0, CHUNK // L, _scale, None)
            for j in range(SBC):
                pltpu.sync_copy(gath_v.at[pl.ds(j * SB, SB)],
                                acc.at[row_v.at[j]], add=True)
            return carry

        lax.fori_loop(0, CHUNKS_PER_TILE, chunk_body, None)
        plsc.subcore_barrier()
        for z in range(ACC_ROWS_PER_TILE // ZROWS):
            r0 = abase + z * ZROWS
            pltpu.sync_copy(acc.at[pl.ds(r0, ZROWS)],
                            out.at[pl.ds(col_off + r0, ZROWS)])
            if rezero:
                pltpu.sync_copy(zbuf_v, acc.at[pl.ds(r0, ZROWS)])
        plsc.subcore_barrier()

    layer(ego, out1, rezero=True)
    layer(out1, out2, rezero=False)


@jax.jit
def _sc_call(ego_split, cols2, rows2, vals2):
    mesh = plsc.VectorSubcoreMesh(
        core_axis_name="c", subcore_axis_name="s",
        num_cores=NC, num_subcores=NS)
    f = pl.kernel(
        _sc_body,
        out_type=(
            jax.ShapeDtypeStruct((NC * NP, HALF), jnp.float32),
            jax.ShapeDtypeStruct((NC * NP, HALF), jnp.float32),
        ),
        mesh=mesh,
        compiler_params=pltpu.CompilerParams(use_tc_tiling_on_sc=False),
        scratch_types=[
            pltpu.VMEM((CHUNK,), jnp.int32),
            pltpu.VMEM((CHUNK,), jnp.int32),
            pltpu.VMEM((SBC, SB), jnp.int32),
            pltpu.VMEM((SBC, SB), jnp.int32),
            pltpu.VMEM((CHUNK,), jnp.float32),
            pltpu.VMEM((CHUNK, HALF), jnp.float32),
            pltpu.VMEM((ZROWS, HALF), jnp.float32),
            pltpu.VMEM_SHARED((NP, HALF), jnp.float32),
            pltpu.SemaphoreType.DMA,
        ],
    )
    return f(ego_split, cols2, rows2, vals2)


def kernel(user_emb, item_emb, hg_rows, hg_cols, hg_vals):
    ego = jnp.concatenate([user_emb, item_emb], axis=0)
    pad = jnp.zeros((NP - N, HALF), jnp.float32)
    ego_split = jnp.concatenate(
        [ego[:, :HALF], pad, ego[:, HALF:], pad], axis=0)
    cols2 = hg_cols
    rows2 = hg_rows
    vals2 = hg_vals
    out1, out2 = _sc_call(ego_split, cols2, rows2, vals2)
    e1 = jnp.concatenate([out1[:N], out1[NP:NP + N]], axis=1)
    e2 = jnp.concatenate([out2[:N], out2[NP:NP + N]], axis=1)
    all_emb = jnp.concatenate([ego, e1, e2], axis=1)
    return all_emb[:N_USER], all_emb[N_USER:]


# speedup vs baseline: 11.3597x; 1.9581x over previous
import jax
import jax.numpy as jnp
from jax import lax
from jax.experimental import pallas as pl
from jax.experimental.pallas import tpu as pltpu
from jax.experimental.pallas import tpu_sc as plsc

N_USER = 20000
N_ITEM = 30000
N = N_USER + N_ITEM
NP = 51200
D = 64
HALF = 32
E = 800000
NC = 2
NS = 16
L = 16

SB = 80
SBC = 5
CHUNK = SB * SBC
EDGES_PER_TILE = E // NS
NCHUNK = EDGES_PER_TILE // CHUNK
ACC_ROWS_PER_TILE = NP // NS
FLUSH_STEPS = ACC_ROWS_PER_TILE // CHUNK


def _sc_body(ego, cols, rows, vals, out1, out2,
             cidx, crow, cval, gath, acc, semI, semG, semS):
    c = lax.axis_index("c")
    s = lax.axis_index("s")
    col_off = c * NP
    ebase = s * EDGES_PER_TILE

    z16 = jnp.zeros((L,), jnp.float32)

    def _zero_slab(r, carry):
        gath[0, r, pl.ds(0, L)] = z16
        gath[0, r, pl.ds(L, L)] = z16
        return carry

    lax.fori_loop(0, CHUNK, _zero_slab, None)

    abase = s * ACC_ROWS_PER_TILE
    for z in range(FLUSH_STEPS):
        pltpu.sync_copy(gath.at[0], acc.at[pl.ds(abase + z * CHUNK, CHUNK)])
    plsc.subcore_barrier()

    def fire_index(g, carry_done=None):
        e0 = ebase + g * CHUNK
        sb0 = e0 // SB
        pltpu.async_copy(cols.at[pl.ds(sb0, SBC)], cidx.at[lax.rem(g, 2)], semI)
        pltpu.async_copy(rows.at[pl.ds(sb0, SBC)], crow.at[lax.rem(g, 3)], semI)
        pltpu.async_copy(vals.at[pl.ds(e0, CHUNK)], cval.at[lax.rem(g, 2)], semI)

    def wait_index(g):
        e0 = ebase + g * CHUNK
        sb0 = e0 // SB
        pltpu.make_async_copy(
            cols.at[pl.ds(sb0, SBC)], cidx.at[lax.rem(g, 2)], semI).wait()
        pltpu.make_async_copy(
            rows.at[pl.ds(sb0, SBC)], crow.at[lax.rem(g, 3)], semI).wait()
        pltpu.make_async_copy(
            vals.at[pl.ds(e0, CHUNK)], cval.at[lax.rem(g, 2)], semI).wait()

    def adjust(g):
        p = lax.rem(g, 2)
        for j in range(SBC):
            for k in range(SB // L):
                sl = pl.ds(k * L, L)
                cidx[p, j, sl] = cidx[p, j, sl] + col_off

    def fire_gathers(table, g):
        p = lax.rem(g, 2)
        for j in range(SBC):
            pltpu.async_copy(table.at[cidx.at[p, j]],
                             gath.at[p, pl.ds(j * SB, SB)], semG)

    def drain_gathers(table, g):
        p = lax.rem(g, 2)
        for j in range(SBC):
            pltpu.make_async_copy(table.at[cidx.at[p, j]],
                                  gath.at[p, pl.ds(j * SB, SB)], semG).wait()

    def scale(g):
        p = lax.rem(g, 2)

        def _scale(t, cy):
            valvec = cval[p, pl.ds(t * L, L)]
            for i in range(L):
                r = t * L + i
                v = valvec[i]
                gath[p, r, pl.ds(0, L)] = gath[p, r, pl.ds(0, L)] * v
                gath[p, r, pl.ds(L, L)] = gath[p, r, pl.ds(L, L)] * v
            return cy

        lax.fori_loop(0, CHUNK // L, _scale, None)

    def fire_scatters(g):
        p = lax.rem(g, 2)
        pi = lax.rem(g, 3)
        for j in range(SBC):
            pltpu.async_copy(gath.at[p, pl.ds(j * SB, SB)],
                             acc.at[crow.at[pi, j]], semS, add=True)

    def drain_scatters(g):
        p = lax.rem(g, 2)
        pi = lax.rem(g, 3)
        for j in range(SBC):
            pltpu.make_async_copy(gath.at[p, pl.ds(j * SB, SB)],
                                  acc.at[crow.at[pi, j]], semS).wait()

    def layer(table, out, rezero):
        fire_index(0)
        wait_index(0)
        adjust(0)
        fire_gathers(table, 0)
        fire_index(1)

        def chunk_body(g, carry):
            drain_gathers(table, g)

            @pl.when(g + 1 < NCHUNK)
            def _():
                wait_index(g + 1)
                adjust(g + 1)

                @pl.when(g >= 1)
                def _():
                    drain_scatters(g - 1)

                fire_gathers(table, g + 1)

            scale(g)
            fire_scatters(g)

            @pl.when(g + 2 < NCHUNK)
            def _():
                fire_index(g + 2)
            return carry

        lax.fori_loop(0, NCHUNK, chunk_body, None)
        drain_scatters(NCHUNK - 2)
        drain_scatters(NCHUNK - 1)
        plsc.subcore_barrier()
        if rezero:
            lax.fori_loop(0, CHUNK, _zero_slab, None)
        for z in range(FLUSH_STEPS):
            r0 = abase + z * CHUNK
            pltpu.sync_copy(acc.at[pl.ds(r0, CHUNK)],
                            out.at[pl.ds(col_off + r0, CHUNK)])
            if rezero:
                pltpu.sync_copy(gath.at[0], acc.at[pl.ds(r0, CHUNK)])
        plsc.subcore_barrier()

    layer(ego, out1, rezero=True)
    layer(out1, out2, rezero=False)


@jax.jit
def _sc_call(ego_split, cols2, rows2, vals2):
    mesh = plsc.VectorSubcoreMesh(
        core_axis_name="c", subcore_axis_name="s",
        num_cores=NC, num_subcores=NS)
    f = pl.kernel(
        _sc_body,
        out_type=(
            jax.ShapeDtypeStruct((NC * NP, HALF), jnp.float32),
            jax.ShapeDtypeStruct((NC * NP, HALF), jnp.float32),
        ),
        mesh=mesh,
        compiler_params=pltpu.CompilerParams(use_tc_tiling_on_sc=False),
        scratch_types=[
            pltpu.VMEM((2, SBC, SB), jnp.int32),
            pltpu.VMEM((3, SBC, SB), jnp.int32),
            pltpu.VMEM((2, CHUNK), jnp.float32),
            pltpu.VMEM((2, CHUNK, HALF), jnp.float32),
            pltpu.VMEM_SHARED((NP, HALF), jnp.float32),
            pltpu.SemaphoreType.DMA,
            pltpu.SemaphoreType.DMA,
            pltpu.SemaphoreType.DMA,
        ],
    )
    return f(ego_split, cols2, rows2, vals2)


def kernel(user_emb, item_emb, hg_rows, hg_cols, hg_vals):
    ego = jnp.concatenate([user_emb, item_emb], axis=0)
    pad = jnp.zeros((NP - N, HALF), jnp.float32)
    ego_split = jnp.concatenate(
        [ego[:, :HALF], pad, ego[:, HALF:], pad], axis=0)
    cols2 = hg_cols.reshape(E // SB, SB)
    rows2 = hg_rows.reshape(E // SB, SB)
    vals2 = hg_vals
    out1, out2 = _sc_call(ego_split, cols2, rows2, vals2)
    e1 = jnp.concatenate([out1[:N], out1[NP:NP + N]], axis=1)
    e2 = jnp.concatenate([out2[:N], out2[NP:NP + N]], axis=1)
    all_emb = jnp.concatenate([ego, e1, e2], axis=1)
    return all_emb[:N_USER], all_emb[N_USER:]
